# baseline (speedup 1.0000x reference)
import jax
import jax.numpy as jnp
from jax.experimental import pallas as pl
from jax.experimental.pallas import tpu as pltpu

N_LABELS = 10000
N_CLS = 128
N_CTX = 16
CTX_DIM = 512
SEQ_LEN = 77
BATCH = 1024
N_SUF = SEQ_LEN - 1 - N_CTX


def _assemble(lab_ref, map_ref, pref_ref, ctx_ref, suf_ref, out_ref):
    del lab_ref, map_ref
    out_ref[:, pl.ds(0, 1)] = pref_ref[:]
    out_ref[:, pl.ds(1, N_CTX)] = ctx_ref[:]
    out_ref[:, pl.ds(1 + N_CTX, N_SUF)] = suf_ref[:]


_grid_spec = pltpu.PrefetchScalarGridSpec(
    num_scalar_prefetch=2,
    grid=(BATCH,),
    in_specs=[
        pl.BlockSpec((1, 1, CTX_DIM),
                     lambda i, lab, mp: (lab[i], 0, 0)),
        pl.BlockSpec((1, N_CTX, CTX_DIM),
                     lambda i, lab, mp: (mp[lab[i]], 0, 0)),
        pl.BlockSpec((1, N_SUF, CTX_DIM),
                     lambda i, lab, mp: (lab[i], 0, 0)),
    ],
    out_specs=pl.BlockSpec((1, SEQ_LEN, CTX_DIM),
                           lambda i, lab, mp: (i, 0, 0)),
)

_prompt_gather = pl.pallas_call(
    _assemble,
    grid_spec=_grid_spec,
    out_shape=jax.ShapeDtypeStruct((BATCH, SEQ_LEN, CTX_DIM), jnp.float32),
)


def kernel(label_ids, mapping, ctx, token_prefix, token_suffix):
    lab = label_ids.astype(jnp.int32)
    return _prompt_gather(lab, mapping, token_prefix, ctx, token_suffix)


# speedup vs baseline: 1.3439x; 1.3439x over previous
import jax
import jax.numpy as jnp
from jax.experimental import pallas as pl
from jax.experimental.pallas import tpu as pltpu

N_LABELS = 10000
N_CLS = 128
N_CTX = 16
CTX_DIM = 512
SEQ_LEN = 77
BATCH = 1024
N_SUF = SEQ_LEN - 1 - N_CTX
IPS = 8


def _assemble(lab_ref, map_ref, *refs):
    del lab_ref, map_ref
    pref_refs = refs[0:IPS]
    ctx_refs = refs[IPS:2 * IPS]
    suf_refs = refs[2 * IPS:3 * IPS]
    out_ref = refs[3 * IPS]
    for j in range(IPS):
        row = pl.ds(j, 1)
        out_ref[row, pl.ds(0, 1)] = pref_refs[j][:]
        out_ref[row, pl.ds(1, N_CTX)] = ctx_refs[j][:]
        out_ref[row, pl.ds(1 + N_CTX, N_SUF)] = suf_refs[j][:]


def _pref_spec(j):
    return pl.BlockSpec((1, 1, CTX_DIM),
                        lambda i, lab, mp: (lab[i * IPS + j], 0, 0))


def _ctx_spec(j):
    return pl.BlockSpec((1, N_CTX, CTX_DIM),
                        lambda i, lab, mp: (mp[lab[i * IPS + j]], 0, 0))


def _suf_spec(j):
    return pl.BlockSpec((1, N_SUF, CTX_DIM),
                        lambda i, lab, mp: (lab[i * IPS + j], 0, 0))


_grid_spec = pltpu.PrefetchScalarGridSpec(
    num_scalar_prefetch=2,
    grid=(BATCH // IPS,),
    in_specs=([_pref_spec(j) for j in range(IPS)]
              + [_ctx_spec(j) for j in range(IPS)]
              + [_suf_spec(j) for j in range(IPS)]),
    out_specs=pl.BlockSpec((IPS, SEQ_LEN, CTX_DIM),
                           lambda i, lab, mp: (i, 0, 0)),
)

_prompt_gather = pl.pallas_call(
    _assemble,
    grid_spec=_grid_spec,
    out_shape=jax.ShapeDtypeStruct((BATCH, SEQ_LEN, CTX_DIM), jnp.float32),
)


def kernel(label_ids, mapping, ctx, token_prefix, token_suffix):
    lab = label_ids.astype(jnp.int32)
    return _prompt_gather(lab, mapping,
                          *([token_prefix] * IPS
                            + [ctx] * IPS
                            + [token_suffix] * IPS))


# speedup vs baseline: 1.3480x; 1.0030x over previous
import functools

import jax
import jax.numpy as jnp
from jax import lax
from jax.experimental import pallas as pl
from jax.experimental.pallas import tpu as pltpu
from jax.experimental.pallas import tpu_sc as plsc

N_LABELS = 10000
N_CLS = 128
N_CTX = 16
CTX_DIM = 512
SEQ_LEN = 77
BATCH = 1024
N_SUF = SEQ_LEN - 1 - N_CTX

NC, NS = 2, 16
NW = NC * NS
BPW = BATCH // NW
LANES = 16
CHUNK = 24
TAIL = SEQ_LEN - 3 * CHUNK


def _copy_rows(copies):
    def body(c, carry):
        sl = pl.ds(pl.multiple_of(c * LANES, 8), LANES)
        for src, soff, dst, doff, n_rows in copies:
            for j in range(n_rows):
                dst[j + doff, sl] = src[j + soff, sl]
        return carry
    lax.fori_loop(0, CTX_DIM // LANES, body, 0)


@functools.partial(
    pl.kernel,
    out_type=jax.ShapeDtypeStruct((BATCH, SEQ_LEN, CTX_DIM), jnp.float32),
    mesh=plsc.VectorSubcoreMesh(core_axis_name="c", subcore_axis_name="s"),
    scratch_types=[
        pltpu.VMEM((BPW,), jnp.int32),
        pltpu.VMEM((N_LABELS,), jnp.int32),
        pltpu.SMEM((BPW,), jnp.int32),
        pltpu.SMEM((LANES,), jnp.int32),
        pltpu.VMEM((1, CTX_DIM), jnp.float32),
        pltpu.VMEM((1, CTX_DIM), jnp.float32),
        pltpu.VMEM((N_CTX, CTX_DIM), jnp.float32),
        pltpu.VMEM((N_CTX, CTX_DIM), jnp.float32),
        pltpu.VMEM((N_SUF, CTX_DIM), jnp.float32),
        pltpu.VMEM((N_SUF, CTX_DIM), jnp.float32),
        pltpu.VMEM((CHUNK, CTX_DIM), jnp.float32),
        pltpu.VMEM((CHUNK, CTX_DIM), jnp.float32),
        pltpu.SemaphoreType.DMA,
        pltpu.SemaphoreType.DMA,
        pltpu.SemaphoreType.DMA,
        pltpu.SemaphoreType.DMA,
    ],
)
def _prompt_gather(label_hbm, map_hbm, ctx_hbm, pref_hbm, suf_hbm,
                   out_hbm, lab_v, map_v, labs, msmem, p0, p1, c0, c1,
                   s0, s1, kb0, kb1, g0, g1, w0, w1):
    pstage = (p0, p1)
    cstage = (c0, c1)
    sstage = (s0, s1)
    gsem = (g0, g1)

    wid = lax.axis_index("s") * NC + lax.axis_index("c")
    base = wid * BPW

    pltpu.sync_copy(label_hbm.at[pl.ds(base, BPW)], lab_v)
    pltpu.sync_copy(map_hbm, map_v)
    for half in range(BPW // LANES):
        lv = lab_v[pl.ds(half * LANES, LANES)]
        for k in range(LANES):
            labs[half * LANES + k] = lv[k]

    def issue_gathers(i, sp):
        lab = labs[i]
        moff = lab % LANES
        mv = map_v[pl.ds(pl.multiple_of(lab - moff, 8), LANES)]
        for k in range(LANES):
            msmem[k] = mv[k]
        cid = msmem[moff]
        pltpu.async_copy(pref_hbm.at[lab], pstage[sp], gsem[sp])
        pltpu.async_copy(ctx_hbm.at[cid], cstage[sp], gsem[sp])
        pltpu.async_copy(suf_hbm.at[lab], sstage[sp], gsem[sp])

    def assemble(i, sp):
        pltpu.make_async_copy(pref_hbm.at[0], pstage[sp], gsem[sp]).wait()
        pltpu.make_async_copy(ctx_hbm.at[0], cstage[sp], gsem[sp]).wait()
        pltpu.make_async_copy(suf_hbm.at[0], sstage[sp], gsem[sp]).wait()
        row = base + i

        @pl.when(i > 0)
        def _():
            pltpu.make_async_copy(
                kb0, out_hbm.at[row, pl.ds(48, CHUNK)], w0).wait()
        _copy_rows([(pstage[sp], 0, kb0, 0, 1),
                    (cstage[sp], 0, kb0, 1, N_CTX),
                    (sstage[sp], 0, kb0, 1 + N_CTX, CHUNK - 1 - N_CTX)])
        h0 = pltpu.async_copy(kb0, out_hbm.at[row, pl.ds(0, CHUNK)], w0)

        @pl.when(i > 0)
        def _():
            pltpu.make_async_copy(
                kb1.at[pl.ds(0, TAIL)],
                out_hbm.at[row, pl.ds(3 * CHUNK, TAIL)], w1).wait()
        _copy_rows([(sstage[sp], CHUNK - 1 - N_CTX, kb1, 0, CHUNK)])
        h1 = pltpu.async_copy(kb1, out_hbm.at[row, pl.ds(CHUNK, CHUNK)], w1)

        h0.wait()
        _copy_rows([(sstage[sp], 2 * CHUNK - 1 - N_CTX, kb0, 0, CHUNK)])
        pltpu.async_copy(kb0, out_hbm.at[row, pl.ds(2 * CHUNK, CHUNK)], w0)

        h1.wait()
        _copy_rows([(sstage[sp], 3 * CHUNK - 1 - N_CTX, kb1, 0, TAIL)])
        pltpu.async_copy(kb1.at[pl.ds(0, TAIL)],
                         out_hbm.at[row, pl.ds(3 * CHUNK, TAIL)], w1)

    issue_gathers(0, 0)

    def pair_body(g, carry):
        issue_gathers(2 * g + 1, 1)
        assemble(2 * g, 0)

        @pl.when(g < BPW // 2 - 1)
        def _():
            issue_gathers(2 * g + 2, 0)
        assemble(2 * g + 1, 1)
        return carry

    lax.fori_loop(0, BPW // 2, pair_body, 0)

    last = base + BPW - 1
    pltpu.make_async_copy(
        kb0, out_hbm.at[last, pl.ds(48, CHUNK)], w0).wait()
    pltpu.make_async_copy(
        kb1.at[pl.ds(0, TAIL)],
        out_hbm.at[last, pl.ds(3 * CHUNK, TAIL)], w1).wait()


def kernel(label_ids, mapping, ctx, token_prefix, token_suffix):
    lab = label_ids.astype(jnp.int32)
    return _prompt_gather(lab, mapping, ctx, token_prefix, token_suffix)
